# baseline (speedup 1.0000x reference)
import functools

import jax
import jax.numpy as jnp
from jax import lax
from jax.experimental import pallas as pl
from jax.experimental.pallas import tpu as pltpu
from jax.experimental.pallas import tpu_sc as plsc

T = 2048
D = 1024
E = 8
FF = 1024
SF = 1024
TK = 2 * T
BLK = 256
NB = TK // BLK
NSTEPS = NB + E - 1
BT2 = 256

_F32 = jnp.float32
_BF16 = jnp.bfloat16
_I32 = jnp.int32


def _router_body(x_ref, gw_ref, bias_ref, pos_ref, p0_ref, p1_ref,
                 w1_ref, w2_ref, meta_ref):
    x = x_ref[...]
    logits = lax.dot_general(
        x, gw_ref[...], (((1,), (1,)), ((), ())),
        preferred_element_type=_F32,
        precision=lax.Precision.DEFAULT,
    )
    scores = jax.nn.sigmoid(logits)
    sfc = scores + bias_ref[...]

    lane = lax.broadcasted_iota(_I32, (T, E), 1)
    big = _F32(1e30)

    m1 = jnp.max(sfc, axis=1, keepdims=True)
    i1 = jnp.min(jnp.where(sfc >= m1, lane, E), axis=1, keepdims=True)
    oh1 = lane == i1
    sfc2 = jnp.where(oh1, -big, sfc)
    m2 = jnp.max(sfc2, axis=1, keepdims=True)
    i2 = jnp.min(jnp.where(sfc2 >= m2, lane, E), axis=1, keepdims=True)
    oh2 = lane == i2

    w1 = jnp.sum(jnp.where(oh1, scores, 0.0), axis=1, keepdims=True)
    w2 = jnp.sum(jnp.where(oh2, scores, 0.0), axis=1, keepdims=True)
    norm = w1 + w2 + 1e-20
    w1 = w1 / norm
    w2 = w2 / norm

    oh_f = oh1.astype(_F32) + oh2.astype(_F32)
    oh_b = oh_f.astype(_BF16)

    CH = 128
    tri = (lax.broadcasted_iota(_I32, (CH, CH), 0)
           > lax.broadcasted_iota(_I32, (CH, CH), 1)).astype(_BF16)
    chunks = []
    running = jnp.zeros((1, E), _F32)
    for c in range(T // CH):
        blk = oh_b[c * CH:(c + 1) * CH, :]
        within = lax.dot_general(
            tri, blk, (((1,), (0,)), ((), ())), preferred_element_type=_F32)
        chunks.append(within + running)
        running = running + jnp.sum(blk.astype(_F32), axis=0, keepdims=True)
    cexcl = jnp.concatenate(chunks, axis=0)
    counts = running

    counts_b = jnp.broadcast_to(counts, (T, E))
    off1 = jnp.sum(jnp.where(lane < i1, counts_b, 0.0), axis=1, keepdims=True)
    off2 = jnp.sum(jnp.where(lane < i2, counts_b, 0.0), axis=1, keepdims=True)
    rank1 = jnp.sum(jnp.where(oh1, cexcl, 0.0), axis=1, keepdims=True)
    rank2 = jnp.sum(jnp.where(oh2, cexcl, 0.0), axis=1, keepdims=True)
    pos1 = (off1 + rank1).astype(_I32)
    pos2 = (off2 + rank2).astype(_I32)

    pos_ref[...] = jnp.concatenate([pos1, pos2], axis=1)
    p0_ref[...] = pos1
    p1_ref[...] = pos2
    w1_ref[...] = w1
    w2_ref[...] = w2

    ones_col = jnp.ones((T, 1), _BF16)
    counts_col = lax.dot_general(
        oh_b, ones_col, (((0,), (0,)), ((), ())),
        preferred_element_type=_F32)
    ltri = (lax.broadcasted_iota(_I32, (E, E), 1)
            < lax.broadcasted_iota(_I32, (E, E), 0)).astype(_BF16)
    ohl = lax.dot_general(
        oh_b, ltri, (((1,), (1,)), ((), ())),
        preferred_element_type=_F32).astype(_BF16)
    offs_col = lax.dot_general(
        ohl, ones_col, (((0,), (0,)), ((), ())),
        preferred_element_type=_F32)

    counts_i = counts_col.astype(_I32)
    offs_i = offs_col.astype(_I32)
    first_b = lax.shift_right_arithmetic(offs_i, 8)
    last_b = lax.shift_right_arithmetic(offs_i + counts_i - 1, 8)
    nsteps = jnp.where(counts_i > 0, last_b - first_b + 1, 0)
    start = lax.dot_general(
        ltri, nsteps.astype(_BF16), (((1,), (0,)), ((), ())),
        preferred_element_type=_F32).astype(_I32)
    total = jnp.sum(nsteps)

    svec = lax.broadcasted_iota(_I32, (E, 128), 1)
    s_eff = jnp.minimum(svec, total - 1)
    start_b2 = jnp.broadcast_to(start, (E, 128))
    e_of_s = jnp.sum((start_b2 <= s_eff).astype(_I32), axis=0,
                     keepdims=True) - 1
    eidx = lax.broadcasted_iota(_I32, (E, 128), 0)
    e_b = jnp.broadcast_to(e_of_s, (E, 128))

    def at_e(col):
        return jnp.sum(jnp.where(eidx == e_b, jnp.broadcast_to(col, (E, 128)),
                                 0), axis=0, keepdims=True)

    start_at = at_e(start)
    first_at = at_e(first_b)
    off_at = at_e(offs_i)
    end_at = at_e(offs_i + counts_i)
    blk_s = first_at + (s_eff[0:1, :] - start_at)
    lo_s = jnp.maximum(off_at, blk_s * BLK)
    hi_s = jnp.minimum(end_at, (blk_s + 1) * BLK)
    valid = svec[0:1, :] < total
    hi_s = jnp.where(valid, hi_s, lo_s)

    meta_ref[...] = jnp.concatenate(
        [e_of_s, blk_s, lo_s, hi_s, jnp.zeros((4, 128), _I32)], axis=0)


def _run_router(x, gate_w, bias2):
    return pl.pallas_call(
        _router_body,
        out_shape=(
            jax.ShapeDtypeStruct((T, 2), _I32),
            jax.ShapeDtypeStruct((T, 1), _I32),
            jax.ShapeDtypeStruct((T, 1), _I32),
            jax.ShapeDtypeStruct((T, 1), _F32),
            jax.ShapeDtypeStruct((T, 1), _F32),
            jax.ShapeDtypeStruct((8, 128), _I32),
        ),
    )(x, gate_w, bias2)


def _make_sc_dispatch():
    mesh = plsc.VectorSubcoreMesh(core_axis_name="c", subcore_axis_name="s")
    NW = 32
    CHUNK = TK // NW
    NSUB = 4
    SUB = CHUNK // NSUB

    @functools.partial(
        pl.kernel, mesh=mesh,
        out_type=jax.ShapeDtypeStruct((TK, D), _F32),
        scratch_types=[
            pltpu.VMEM((SUB,), _I32),
            pltpu.VMEM((SUB,), _I32),
            pltpu.VMEM((SUB,), _I32),
            pltpu.VMEM((SUB,), _I32),
            pltpu.VMEM((SUB, D), _F32),
            pltpu.VMEM((SUB, D), _F32),
            pltpu.SemaphoreType.DMA,
            pltpu.SemaphoreType.DMA,
        ],
    )
    def sc_dispatch(x_hbm, pos_hbm, xs_hbm, pos_b0, pos_b1, tok_b0, tok_b1,
                    rows_b0, rows_b1, semg, sems):
        wid = lax.axis_index("s") * 2 + lax.axis_index("c")
        base = wid * CHUNK
        pos_bufs = [pos_b0, pos_b1]
        tok_bufs = [tok_b0, tok_b1]
        rows_bufs = [rows_b0, rows_b1]

        def prep(k):
            sbase = base + k * SUB
            pv, tv = pos_bufs[k % 2], tok_bufs[k % 2]
            pltpu.sync_copy(pos_hbm.at[pl.ds(sbase, SUB)], pv)
            for j in range(SUB // 16):
                t16 = lax.shift_right_logical(
                    lax.iota(_I32, 16), 1) + ((sbase + 16 * j) // 2)
                tv[pl.ds(16 * j, 16)] = t16

        prep(0)
        g = pltpu.async_copy(x_hbm.at[tok_bufs[0]], rows_bufs[0], semg)
        scat = None
        for k in range(NSUB):
            g.wait()
            if scat is not None:
                scat.wait()
            if k + 1 < NSUB:
                prep(k + 1)
                g = pltpu.async_copy(
                    x_hbm.at[tok_bufs[(k + 1) % 2]],
                    rows_bufs[(k + 1) % 2], semg)
            scat = pltpu.async_copy(
                rows_bufs[k % 2], xs_hbm.at[pos_bufs[k % 2]], sems)
        scat.wait()

    return sc_dispatch


_SC_CACHE = {}


def _get_sc_dispatch():
    if "dispatch" not in _SC_CACHE:
        _SC_CACHE["dispatch"] = _make_sc_dispatch()
    return _SC_CACHE["dispatch"]


def _grouped_body(meta_ref, xs_ref, wgu_ref, wdn_ref, rs_ref, wgu_c, wdn_c):
    s = pl.program_id(0)
    blk = meta_ref[1, s]
    lo = meta_ref[2, s]
    hi = meta_ref[3, s]
    prev = meta_ref[1, jnp.maximum(s - 1, 0)]
    first = jnp.logical_or(s == 0, blk != prev)
    eprev = meta_ref[0, jnp.maximum(s - 1, 0)]
    echanged = jnp.logical_or(s == 0, meta_ref[0, s] != eprev)

    @pl.when(echanged)
    def _():
        wgu_c[...] = wgu_ref[0].astype(_BF16)
        wdn_c[...] = wdn_ref[0].astype(_BF16)

    xb = xs_ref[...].astype(_BF16)
    gu = lax.dot_general(xb, wgu_c[...], (((1,), (1,)), ((), ())),
                         preferred_element_type=_F32)
    g = gu[:, :FF]
    u = gu[:, FF:]
    h = (g * jax.nn.sigmoid(g) * u).astype(_BF16)
    eo = lax.dot_general(h, wdn_c[...], (((1,), (1,)), ((), ())),
                         preferred_element_type=_F32)

    rows = lax.broadcasted_iota(_I32, (BLK, 1), 0) + blk * BLK
    maskv = jnp.logical_and(rows >= lo, rows < hi)
    contrib = jnp.where(maskv, eo, 0.0)

    @pl.when(first)
    def _():
        rs_ref[...] = contrib

    @pl.when(jnp.logical_not(first))
    def _():
        rs_ref[...] = rs_ref[...] + contrib


def _run_grouped(meta, xs, w_gate_up, w_down):
    grid_spec = pltpu.PrefetchScalarGridSpec(
        num_scalar_prefetch=1,
        grid=(NSTEPS,),
        in_specs=[
            pl.BlockSpec((BLK, D), lambda s, m: (m[1, s], 0)),
            pl.BlockSpec((1, 2 * FF, D), lambda s, m: (m[0, s], 0, 0)),
            pl.BlockSpec((1, D, FF), lambda s, m: (m[0, s], 0, 0)),
        ],
        out_specs=pl.BlockSpec((BLK, D), lambda s, m: (m[1, s], 0)),
        scratch_shapes=[
            pltpu.VMEM((2 * FF, D), _BF16),
            pltpu.VMEM((D, FF), _BF16),
        ],
    )
    return pl.pallas_call(
        _grouped_body,
        grid_spec=grid_spec,
        out_shape=jax.ShapeDtypeStruct((TK, D), _F32),
    )(meta, xs, w_gate_up, w_down)


def _make_sc_gather():
    mesh = plsc.VectorSubcoreMesh(core_axis_name="c", subcore_axis_name="s")
    NW = 32
    TPW = T // NW

    SUB = TPW // 2

    @functools.partial(
        pl.kernel, mesh=mesh,
        out_type=(
            jax.ShapeDtypeStruct((T, D), _F32),
            jax.ShapeDtypeStruct((T, D), _F32),
        ),
        scratch_types=[
            pltpu.VMEM((SUB,), _I32),
            pltpu.VMEM((SUB,), _I32),
            pltpu.VMEM((SUB, D), _F32),
            pltpu.VMEM((SUB, D), _F32),
            pltpu.SemaphoreType.DMA,
            pltpu.SemaphoreType.DMA,
        ],
    )
    def sc_gather(rs_hbm, p0_hbm, p1_hbm, a_hbm, b_hbm,
                  idx_b0, idx_b1, rows_b0, rows_b1, semg, sems):
        wid = lax.axis_index("s") * 2 + lax.axis_index("c")
        base = wid * TPW
        idx_bufs = [idx_b0, idx_b1]
        rows_bufs = [rows_b0, rows_b1]
        stages = [
            (p0_hbm, base, a_hbm, base),
            (p0_hbm, base + SUB, a_hbm, base + SUB),
            (p1_hbm, base, b_hbm, base),
            (p1_hbm, base + SUB, b_hbm, base + SUB),
        ]

        def prep(k):
            src, off, _, _ = stages[k]
            pltpu.sync_copy(src.at[pl.ds(off, SUB)], idx_bufs[k % 2])

        prep(0)
        g = pltpu.async_copy(rs_hbm.at[idx_bufs[0]], rows_bufs[0], semg)
        wr = None
        for k in range(4):
            g.wait()
            if wr is not None:
                wr.wait()
            if k + 1 < 4:
                prep(k + 1)
                g = pltpu.async_copy(
                    rs_hbm.at[idx_bufs[(k + 1) % 2]],
                    rows_bufs[(k + 1) % 2], semg)
            _, _, dst, doff = stages[k]
            wr = pltpu.async_copy(
                rows_bufs[k % 2], dst.at[pl.ds(doff, SUB)], sems)
        wr.wait()

    return sc_gather


def _get_sc_gather():
    if "gather" not in _SC_CACHE:
        _SC_CACHE["gather"] = _make_sc_gather()
    return _SC_CACHE["gather"]


def _shared_body(x_ref, sgu_ref, sdn_ref, a_ref, b_ref, w1_ref, w2_ref,
                 out_ref, sgu_c, sdn_c):
    @pl.when(pl.program_id(0) == 0)
    def _():
        sgu_c[...] = sgu_ref[...].astype(_BF16)
        sdn_c[...] = sdn_ref[...].astype(_BF16)

    xb = x_ref[...].astype(_BF16)
    sgu = lax.dot_general(xb, sgu_c[...], (((1,), (1,)), ((), ())),
                          preferred_element_type=_F32)
    sg = sgu[:, :SF]
    su = sgu[:, SF:]
    sh = (sg * jax.nn.sigmoid(sg) * su).astype(_BF16)
    out = lax.dot_general(sh, sdn_c[...], (((1,), (1,)), ((), ())),
                          preferred_element_type=_F32)
    out_ref[...] = out + w1_ref[...] * a_ref[...] + w2_ref[...] * b_ref[...]


def _run_shared(x, shared_gate_up, shared_down, a, b, w1, w2):
    return pl.pallas_call(
        _shared_body,
        grid=(T // BT2,),
        in_specs=[
            pl.BlockSpec((BT2, D), lambda i: (i, 0)),
            pl.BlockSpec((2 * SF, D), lambda i: (0, 0)),
            pl.BlockSpec((D, SF), lambda i: (0, 0)),
            pl.BlockSpec((BT2, D), lambda i: (i, 0)),
            pl.BlockSpec((BT2, D), lambda i: (i, 0)),
            pl.BlockSpec((BT2, 1), lambda i: (i, 0)),
            pl.BlockSpec((BT2, 1), lambda i: (i, 0)),
        ],
        out_specs=pl.BlockSpec((BT2, D), lambda i: (i, 0)),
        out_shape=jax.ShapeDtypeStruct((T, D), _F32),
        scratch_shapes=[
            pltpu.VMEM((2 * SF, D), _BF16),
            pltpu.VMEM((D, SF), _BF16),
        ],
    )(x, shared_gate_up, shared_down, a, b, w1, w2)


def kernel(hidden_states, gate_w, expert_bias, w_gate_up, w_down,
           shared_gate_up, shared_down):
    orig_shape = hidden_states.shape
    x = hidden_states.reshape(-1, orig_shape[-1])
    bias2 = expert_bias.reshape(1, E)

    pos2, p0, p1, w1, w2, meta = _run_router(x, gate_w, bias2)
    pos_flat = pos2.reshape(TK)

    xs = _get_sc_dispatch()(x, pos_flat)

    rs = _run_grouped(meta, xs, w_gate_up, w_down)

    a, b = _get_sc_gather()(rs, p0.reshape(T), p1.reshape(T))

    out = _run_shared(x, shared_gate_up, shared_down, a, b, w1, w2)
    return out.reshape(orig_shape)


# speedup vs baseline: 1.0220x; 1.0220x over previous
import functools

import jax
import jax.numpy as jnp
from jax import lax
from jax.experimental import pallas as pl
from jax.experimental.pallas import tpu as pltpu
from jax.experimental.pallas import tpu_sc as plsc

T = 2048
D = 1024
E = 8
FF = 1024
SF = 1024
TK = 2 * T
BLK = 256
NB = TK // BLK
NSTEPS = NB + E - 1
BT2 = 256

_F32 = jnp.float32
_BF16 = jnp.bfloat16
_I32 = jnp.int32


def _router_body(x_ref, gw_ref, bias_ref, pos_ref, p0_ref, p1_ref,
                 w1_ref, w2_ref, meta_ref):
    x = x_ref[...]
    logits = lax.dot_general(
        x, gw_ref[...], (((1,), (1,)), ((), ())),
        preferred_element_type=_F32,
        precision=lax.Precision.DEFAULT,
    )
    scores = jax.nn.sigmoid(logits)
    sfc = scores + bias_ref[...]

    lane = lax.broadcasted_iota(_I32, (T, E), 1)
    big = _F32(1e30)

    m1 = jnp.max(sfc, axis=1, keepdims=True)
    i1 = jnp.min(jnp.where(sfc >= m1, lane, E), axis=1, keepdims=True)
    oh1 = lane == i1
    sfc2 = jnp.where(oh1, -big, sfc)
    m2 = jnp.max(sfc2, axis=1, keepdims=True)
    i2 = jnp.min(jnp.where(sfc2 >= m2, lane, E), axis=1, keepdims=True)
    oh2 = lane == i2

    w1 = jnp.sum(jnp.where(oh1, scores, 0.0), axis=1, keepdims=True)
    w2 = jnp.sum(jnp.where(oh2, scores, 0.0), axis=1, keepdims=True)
    norm = w1 + w2 + 1e-20
    w1 = w1 / norm
    w2 = w2 / norm

    oh_f = oh1.astype(_F32) + oh2.astype(_F32)
    oh_b = oh_f.astype(_BF16)

    CH = 128
    tri = (lax.broadcasted_iota(_I32, (CH, CH), 0)
           > lax.broadcasted_iota(_I32, (CH, CH), 1)).astype(_BF16)
    chunks = []
    running = jnp.zeros((1, E), _F32)
    for c in range(T // CH):
        blk = oh_b[c * CH:(c + 1) * CH, :]
        within = lax.dot_general(
            tri, blk, (((1,), (0,)), ((), ())), preferred_element_type=_F32)
        chunks.append(within + running)
        running = running + jnp.sum(blk.astype(_F32), axis=0, keepdims=True)
    cexcl = jnp.concatenate(chunks, axis=0)
    counts = running

    counts_b = jnp.broadcast_to(counts, (T, E))
    off1 = jnp.sum(jnp.where(lane < i1, counts_b, 0.0), axis=1, keepdims=True)
    off2 = jnp.sum(jnp.where(lane < i2, counts_b, 0.0), axis=1, keepdims=True)
    rank1 = jnp.sum(jnp.where(oh1, cexcl, 0.0), axis=1, keepdims=True)
    rank2 = jnp.sum(jnp.where(oh2, cexcl, 0.0), axis=1, keepdims=True)
    pos1 = (off1 + rank1).astype(_I32)
    pos2 = (off2 + rank2).astype(_I32)

    pos_ref[...] = jnp.concatenate([pos1, pos2], axis=1)
    p0_ref[...] = pos1
    p1_ref[...] = pos2
    w1_ref[...] = w1
    w2_ref[...] = w2

    ones_col = jnp.ones((T, 1), _BF16)
    counts_col = lax.dot_general(
        oh_b, ones_col, (((0,), (0,)), ((), ())),
        preferred_element_type=_F32)
    ltri = (lax.broadcasted_iota(_I32, (E, E), 1)
            < lax.broadcasted_iota(_I32, (E, E), 0)).astype(_BF16)
    ohl = lax.dot_general(
        oh_b, ltri, (((1,), (1,)), ((), ())),
        preferred_element_type=_F32).astype(_BF16)
    offs_col = lax.dot_general(
        ohl, ones_col, (((0,), (0,)), ((), ())),
        preferred_element_type=_F32)

    counts_i = counts_col.astype(_I32)
    offs_i = offs_col.astype(_I32)
    first_b = lax.shift_right_arithmetic(offs_i, 8)
    last_b = lax.shift_right_arithmetic(offs_i + counts_i - 1, 8)
    nsteps = jnp.where(counts_i > 0, last_b - first_b + 1, 0)
    start = lax.dot_general(
        ltri, nsteps.astype(_BF16), (((1,), (0,)), ((), ())),
        preferred_element_type=_F32).astype(_I32)
    total = jnp.sum(nsteps)

    svec = lax.broadcasted_iota(_I32, (E, 128), 1)
    s_eff = jnp.minimum(svec, total - 1)
    start_b2 = jnp.broadcast_to(start, (E, 128))
    e_of_s = jnp.sum((start_b2 <= s_eff).astype(_I32), axis=0,
                     keepdims=True) - 1
    eidx = lax.broadcasted_iota(_I32, (E, 128), 0)
    e_b = jnp.broadcast_to(e_of_s, (E, 128))

    def at_e(col):
        return jnp.sum(jnp.where(eidx == e_b, jnp.broadcast_to(col, (E, 128)),
                                 0), axis=0, keepdims=True)

    start_at = at_e(start)
    first_at = at_e(first_b)
    off_at = at_e(offs_i)
    end_at = at_e(offs_i + counts_i)
    blk_s = first_at + (s_eff[0:1, :] - start_at)
    lo_s = jnp.maximum(off_at, blk_s * BLK)
    hi_s = jnp.minimum(end_at, (blk_s + 1) * BLK)
    valid = svec[0:1, :] < total
    hi_s = jnp.where(valid, hi_s, lo_s)

    meta_ref[...] = jnp.concatenate(
        [e_of_s, blk_s, lo_s, hi_s, jnp.zeros((4, 128), _I32)], axis=0)


def _run_router(x, gate_w, bias2):
    return pl.pallas_call(
        _router_body,
        out_shape=(
            jax.ShapeDtypeStruct((T, 2), _I32),
            jax.ShapeDtypeStruct((T, 1), _I32),
            jax.ShapeDtypeStruct((T, 1), _I32),
            jax.ShapeDtypeStruct((T, 1), _F32),
            jax.ShapeDtypeStruct((T, 1), _F32),
            jax.ShapeDtypeStruct((8, 128), _I32),
        ),
    )(x, gate_w, bias2)


def _make_sc_dispatch():
    mesh = plsc.VectorSubcoreMesh(core_axis_name="c", subcore_axis_name="s")
    NW = 32
    CHUNK = TK // NW
    NSUB = 4
    SUB = CHUNK // NSUB

    @functools.partial(
        pl.kernel, mesh=mesh,
        out_type=jax.ShapeDtypeStruct((TK, D), _F32),
        scratch_types=[
            pltpu.VMEM((SUB,), _I32),
            pltpu.VMEM((SUB,), _I32),
            pltpu.VMEM((SUB,), _I32),
            pltpu.VMEM((SUB,), _I32),
            pltpu.VMEM((SUB, D), _F32),
            pltpu.VMEM((SUB, D), _F32),
            pltpu.SemaphoreType.DMA,
            pltpu.SemaphoreType.DMA,
        ],
    )
    def sc_dispatch(x_hbm, pos_hbm, xs_hbm, pos_b0, pos_b1, tok_b0, tok_b1,
                    rows_b0, rows_b1, semg, sems):
        wid = lax.axis_index("s") * 2 + lax.axis_index("c")
        base = wid * CHUNK
        pos_bufs = [pos_b0, pos_b1]
        tok_bufs = [tok_b0, tok_b1]
        rows_bufs = [rows_b0, rows_b1]

        def prep(k):
            sbase = base + k * SUB
            pv, tv = pos_bufs[k % 2], tok_bufs[k % 2]
            pltpu.sync_copy(pos_hbm.at[pl.ds(sbase, SUB)], pv)
            for j in range(SUB // 16):
                t16 = lax.shift_right_logical(
                    lax.iota(_I32, 16), 1) + ((sbase + 16 * j) // 2)
                tv[pl.ds(16 * j, 16)] = t16

        prep(0)
        g = pltpu.async_copy(x_hbm.at[tok_bufs[0]], rows_bufs[0], semg)
        scat = None
        for k in range(NSUB):
            g.wait()
            if scat is not None:
                scat.wait()
            if k + 1 < NSUB:
                prep(k + 1)
                g = pltpu.async_copy(
                    x_hbm.at[tok_bufs[(k + 1) % 2]],
                    rows_bufs[(k + 1) % 2], semg)
            scat = pltpu.async_copy(
                rows_bufs[k % 2], xs_hbm.at[pos_bufs[k % 2]], sems)
        scat.wait()

    return sc_dispatch


_SC_CACHE = {}


def _get_sc_dispatch():
    if "dispatch" not in _SC_CACHE:
        _SC_CACHE["dispatch"] = _make_sc_dispatch()
    return _SC_CACHE["dispatch"]


def _grouped_body(meta_ref, xs_ref, wgu_ref, wdn_ref, rs_ref):
    s = pl.program_id(0)
    blk = meta_ref[1, s]
    lo = meta_ref[2, s]
    hi = meta_ref[3, s]
    prev = meta_ref[1, jnp.maximum(s - 1, 0)]
    first = jnp.logical_or(s == 0, blk != prev)
    xb = xs_ref[...].astype(_BF16)
    wgu = wgu_ref[0].astype(_BF16)
    gu = lax.dot_general(xb, wgu, (((1,), (1,)), ((), ())),
                         preferred_element_type=_F32)
    g = gu[:, :FF]
    u = gu[:, FF:]
    h = (g * jax.nn.sigmoid(g) * u).astype(_BF16)
    wdn = wdn_ref[0].astype(_BF16)
    eo = lax.dot_general(h, wdn, (((1,), (1,)), ((), ())),
                         preferred_element_type=_F32)

    rows = lax.broadcasted_iota(_I32, (BLK, 1), 0) + blk * BLK
    maskv = jnp.logical_and(rows >= lo, rows < hi)
    contrib = jnp.where(maskv, eo, 0.0)

    @pl.when(first)
    def _():
        rs_ref[...] = contrib

    @pl.when(jnp.logical_not(first))
    def _():
        rs_ref[...] = rs_ref[...] + contrib


def _run_grouped(meta, xs, w_gate_up, w_down):
    grid_spec = pltpu.PrefetchScalarGridSpec(
        num_scalar_prefetch=1,
        grid=(NSTEPS,),
        in_specs=[
            pl.BlockSpec((BLK, D), lambda s, m: (m[1, s], 0)),
            pl.BlockSpec((1, 2 * FF, D), lambda s, m: (m[0, s], 0, 0)),
            pl.BlockSpec((1, D, FF), lambda s, m: (m[0, s], 0, 0)),
        ],
        out_specs=pl.BlockSpec((BLK, D), lambda s, m: (m[1, s], 0)),
    )
    return pl.pallas_call(
        _grouped_body,
        grid_spec=grid_spec,
        out_shape=jax.ShapeDtypeStruct((TK, D), _F32),
    )(meta, xs, w_gate_up, w_down)


def _make_sc_gather():
    mesh = plsc.VectorSubcoreMesh(core_axis_name="c", subcore_axis_name="s")
    NW = 32
    TPW = T // NW

    SUB = TPW // 2

    @functools.partial(
        pl.kernel, mesh=mesh,
        out_type=(
            jax.ShapeDtypeStruct((T, D), _F32),
            jax.ShapeDtypeStruct((T, D), _F32),
        ),
        scratch_types=[
            pltpu.VMEM((SUB,), _I32),
            pltpu.VMEM((SUB,), _I32),
            pltpu.VMEM((SUB, D), _F32),
            pltpu.VMEM((SUB, D), _F32),
            pltpu.SemaphoreType.DMA,
            pltpu.SemaphoreType.DMA,
        ],
    )
    def sc_gather(rs_hbm, p0_hbm, p1_hbm, a_hbm, b_hbm,
                  idx_b0, idx_b1, rows_b0, rows_b1, semg, sems):
        wid = lax.axis_index("s") * 2 + lax.axis_index("c")
        base = wid * TPW
        idx_bufs = [idx_b0, idx_b1]
        rows_bufs = [rows_b0, rows_b1]
        stages = [
            (p0_hbm, base, a_hbm, base),
            (p0_hbm, base + SUB, a_hbm, base + SUB),
            (p1_hbm, base, b_hbm, base),
            (p1_hbm, base + SUB, b_hbm, base + SUB),
        ]

        def prep(k):
            src, off, _, _ = stages[k]
            pltpu.sync_copy(src.at[pl.ds(off, SUB)], idx_bufs[k % 2])

        prep(0)
        g = pltpu.async_copy(rs_hbm.at[idx_bufs[0]], rows_bufs[0], semg)
        wr = None
        for k in range(4):
            g.wait()
            if wr is not None:
                wr.wait()
            if k + 1 < 4:
                prep(k + 1)
                g = pltpu.async_copy(
                    rs_hbm.at[idx_bufs[(k + 1) % 2]],
                    rows_bufs[(k + 1) % 2], semg)
            _, _, dst, doff = stages[k]
            wr = pltpu.async_copy(
                rows_bufs[k % 2], dst.at[pl.ds(doff, SUB)], sems)
        wr.wait()

    return sc_gather


def _get_sc_gather():
    if "gather" not in _SC_CACHE:
        _SC_CACHE["gather"] = _make_sc_gather()
    return _SC_CACHE["gather"]


def _shared_body(x_ref, sgu_ref, sdn_ref, a_ref, b_ref, w1_ref, w2_ref,
                 out_ref, sgu_c, sdn_c):
    @pl.when(pl.program_id(0) == 0)
    def _():
        sgu_c[...] = sgu_ref[...].astype(_BF16)
        sdn_c[...] = sdn_ref[...].astype(_BF16)

    xb = x_ref[...].astype(_BF16)
    sgu = lax.dot_general(xb, sgu_c[...], (((1,), (1,)), ((), ())),
                          preferred_element_type=_F32)
    sg = sgu[:, :SF]
    su = sgu[:, SF:]
    sh = (sg * jax.nn.sigmoid(sg) * su).astype(_BF16)
    out = lax.dot_general(sh, sdn_c[...], (((1,), (1,)), ((), ())),
                          preferred_element_type=_F32)
    out_ref[...] = out + w1_ref[...] * a_ref[...] + w2_ref[...] * b_ref[...]


def _run_shared(x, shared_gate_up, shared_down, a, b, w1, w2):
    return pl.pallas_call(
        _shared_body,
        grid=(T // BT2,),
        in_specs=[
            pl.BlockSpec((BT2, D), lambda i: (i, 0)),
            pl.BlockSpec((2 * SF, D), lambda i: (0, 0)),
            pl.BlockSpec((D, SF), lambda i: (0, 0)),
            pl.BlockSpec((BT2, D), lambda i: (i, 0)),
            pl.BlockSpec((BT2, D), lambda i: (i, 0)),
            pl.BlockSpec((BT2, 1), lambda i: (i, 0)),
            pl.BlockSpec((BT2, 1), lambda i: (i, 0)),
        ],
        out_specs=pl.BlockSpec((BT2, D), lambda i: (i, 0)),
        out_shape=jax.ShapeDtypeStruct((T, D), _F32),
        scratch_shapes=[
            pltpu.VMEM((2 * SF, D), _BF16),
            pltpu.VMEM((D, SF), _BF16),
        ],
    )(x, shared_gate_up, shared_down, a, b, w1, w2)


def kernel(hidden_states, gate_w, expert_bias, w_gate_up, w_down,
           shared_gate_up, shared_down):
    orig_shape = hidden_states.shape
    x = hidden_states.reshape(-1, orig_shape[-1])
    bias2 = expert_bias.reshape(1, E)

    pos2, p0, p1, w1, w2, meta = _run_router(x, gate_w, bias2)
    pos_flat = pos2.reshape(TK)

    xs = _get_sc_dispatch()(x, pos_flat)

    rs = _run_grouped(meta, xs, w_gate_up, w_down)

    a, b = _get_sc_gather()(rs, p0.reshape(T), p1.reshape(T))

    out = _run_shared(x, shared_gate_up, shared_down, a, b, w1, w2)
    return out.reshape(orig_shape)


# speedup vs baseline: 1.0585x; 1.0357x over previous
import functools

import jax
import jax.numpy as jnp
from jax import lax
from jax.experimental import pallas as pl
from jax.experimental.pallas import tpu as pltpu
from jax.experimental.pallas import tpu_sc as plsc

T = 2048
D = 1024
E = 8
FF = 1024
SF = 1024
TK = 2 * T
BLK = 256
NB = TK // BLK
NSTEPS = NB + E - 1
BT2 = 256

_F32 = jnp.float32
_BF16 = jnp.bfloat16
_I32 = jnp.int32


def _router_body(x_ref, gw_ref, bias_ref, pos_ref, p0_ref, p1_ref,
                 w1_ref, w2_ref, meta_ref):
    x = x_ref[...]
    logits = lax.dot_general(
        x, gw_ref[...], (((1,), (1,)), ((), ())),
        preferred_element_type=_F32,
        precision=lax.Precision.DEFAULT,
    )
    scores = jax.nn.sigmoid(logits)
    sfc = scores + bias_ref[...]

    lane = lax.broadcasted_iota(_I32, (T, E), 1)
    big = _F32(1e30)

    m1 = jnp.max(sfc, axis=1, keepdims=True)
    i1 = jnp.min(jnp.where(sfc >= m1, lane, E), axis=1, keepdims=True)
    oh1 = lane == i1
    sfc2 = jnp.where(oh1, -big, sfc)
    m2 = jnp.max(sfc2, axis=1, keepdims=True)
    i2 = jnp.min(jnp.where(sfc2 >= m2, lane, E), axis=1, keepdims=True)
    oh2 = lane == i2

    w1 = jnp.sum(jnp.where(oh1, scores, 0.0), axis=1, keepdims=True)
    w2 = jnp.sum(jnp.where(oh2, scores, 0.0), axis=1, keepdims=True)
    norm = w1 + w2 + 1e-20
    w1 = w1 / norm
    w2 = w2 / norm

    oh_f = oh1.astype(_F32) + oh2.astype(_F32)
    oh_b = oh_f.astype(_BF16)

    CH = 128
    tri = (lax.broadcasted_iota(_I32, (CH, CH), 0)
           > lax.broadcasted_iota(_I32, (CH, CH), 1)).astype(_BF16)
    chunks = []
    running = jnp.zeros((1, E), _F32)
    for c in range(T // CH):
        blk = oh_b[c * CH:(c + 1) * CH, :]
        within = lax.dot_general(
            tri, blk, (((1,), (0,)), ((), ())), preferred_element_type=_F32)
        chunks.append(within + running)
        running = running + jnp.sum(blk.astype(_F32), axis=0, keepdims=True)
    cexcl = jnp.concatenate(chunks, axis=0)
    counts = running

    counts_b = jnp.broadcast_to(counts, (T, E))
    off1 = jnp.sum(jnp.where(lane < i1, counts_b, 0.0), axis=1, keepdims=True)
    off2 = jnp.sum(jnp.where(lane < i2, counts_b, 0.0), axis=1, keepdims=True)
    rank1 = jnp.sum(jnp.where(oh1, cexcl, 0.0), axis=1, keepdims=True)
    rank2 = jnp.sum(jnp.where(oh2, cexcl, 0.0), axis=1, keepdims=True)
    pos1 = (off1 + rank1).astype(_I32)
    pos2 = (off2 + rank2).astype(_I32)

    pos_ref[...] = jnp.concatenate([pos1, pos2], axis=1)
    p0_ref[...] = pos1
    p1_ref[...] = pos2
    w1_ref[...] = w1
    w2_ref[...] = w2

    ones_col = jnp.ones((T, 1), _BF16)
    counts_col = lax.dot_general(
        oh_b, ones_col, (((0,), (0,)), ((), ())),
        preferred_element_type=_F32)
    ltri = (lax.broadcasted_iota(_I32, (E, E), 1)
            < lax.broadcasted_iota(_I32, (E, E), 0)).astype(_BF16)
    ohl = lax.dot_general(
        oh_b, ltri, (((1,), (1,)), ((), ())),
        preferred_element_type=_F32).astype(_BF16)
    offs_col = lax.dot_general(
        ohl, ones_col, (((0,), (0,)), ((), ())),
        preferred_element_type=_F32)

    counts_i = counts_col.astype(_I32)
    offs_i = offs_col.astype(_I32)
    first_b = lax.shift_right_arithmetic(offs_i, 8)
    last_b = lax.shift_right_arithmetic(offs_i + counts_i - 1, 8)
    nsteps = jnp.where(counts_i > 0, last_b - first_b + 1, 0)
    start = lax.dot_general(
        ltri, nsteps.astype(_BF16), (((1,), (0,)), ((), ())),
        preferred_element_type=_F32).astype(_I32)
    total = jnp.sum(nsteps)

    svec = lax.broadcasted_iota(_I32, (E, 128), 1)
    s_eff = jnp.minimum(svec, total - 1)
    start_b2 = jnp.broadcast_to(start, (E, 128))
    e_of_s = jnp.sum((start_b2 <= s_eff).astype(_I32), axis=0,
                     keepdims=True) - 1
    eidx = lax.broadcasted_iota(_I32, (E, 128), 0)
    e_b = jnp.broadcast_to(e_of_s, (E, 128))

    def at_e(col):
        return jnp.sum(jnp.where(eidx == e_b, jnp.broadcast_to(col, (E, 128)),
                                 0), axis=0, keepdims=True)

    start_at = at_e(start)
    first_at = at_e(first_b)
    off_at = at_e(offs_i)
    end_at = at_e(offs_i + counts_i)
    blk_s = first_at + (s_eff[0:1, :] - start_at)
    lo_s = jnp.maximum(off_at, blk_s * BLK)
    hi_s = jnp.minimum(end_at, (blk_s + 1) * BLK)
    valid = svec[0:1, :] < total
    hi_s = jnp.where(valid, hi_s, lo_s)

    meta_ref[...] = jnp.concatenate(
        [e_of_s, blk_s, lo_s, hi_s, jnp.zeros((4, 128), _I32)], axis=0)


def _run_router(x, gate_w, bias2):
    return pl.pallas_call(
        _router_body,
        out_shape=(
            jax.ShapeDtypeStruct((T, 2), _I32),
            jax.ShapeDtypeStruct((T, 1), _I32),
            jax.ShapeDtypeStruct((T, 1), _I32),
            jax.ShapeDtypeStruct((T, 1), _F32),
            jax.ShapeDtypeStruct((T, 1), _F32),
            jax.ShapeDtypeStruct((8, 128), _I32),
        ),
    )(x, gate_w, bias2)


def _make_sc_dispatch():
    mesh = plsc.VectorSubcoreMesh(core_axis_name="c", subcore_axis_name="s")
    NW = 32
    CHUNK = TK // NW
    SUB = CHUNK // 2

    @functools.partial(
        pl.kernel, mesh=mesh,
        out_type=jax.ShapeDtypeStruct((TK, D), _F32),
        scratch_types=[
            pltpu.VMEM((SUB,), _I32),
            pltpu.VMEM((SUB,), _I32),
            pltpu.VMEM((SUB, D), _F32),
            pltpu.SemaphoreType.DMA,
        ],
    )
    def sc_dispatch(x_hbm, pos_hbm, xs_hbm, pos_v, tok_v, rows_v, sem):
        wid = lax.axis_index("s") * 2 + lax.axis_index("c")
        base = wid * CHUNK
        for sub in range(2):
            sbase = base + sub * SUB
            pltpu.sync_copy(pos_hbm.at[pl.ds(sbase, SUB)], pos_v)
            for j in range(SUB // 16):
                t16 = lax.shift_right_logical(
                    lax.iota(_I32, 16), 1) + ((sbase + 16 * j) // 2)
                tok_v[pl.ds(16 * j, 16)] = t16
            pltpu.async_copy(x_hbm.at[tok_v], rows_v, sem).wait()
            pltpu.async_copy(rows_v, xs_hbm.at[pos_v], sem).wait()

    return sc_dispatch


_SC_CACHE = {}


def _get_sc_dispatch():
    if "dispatch" not in _SC_CACHE:
        _SC_CACHE["dispatch"] = _make_sc_dispatch()
    return _SC_CACHE["dispatch"]


def _grouped_body(meta_ref, xs_ref, wgu_ref, wdn_ref, rs_ref):
    s = pl.program_id(0)
    blk = meta_ref[1, s]
    lo = meta_ref[2, s]
    hi = meta_ref[3, s]
    prev = meta_ref[1, jnp.maximum(s - 1, 0)]
    first = jnp.logical_or(s == 0, blk != prev)
    xb = xs_ref[...].astype(_BF16)
    wgu = wgu_ref[0].astype(_BF16)
    gu = lax.dot_general(xb, wgu, (((1,), (1,)), ((), ())),
                         preferred_element_type=_F32)
    g = gu[:, :FF]
    u = gu[:, FF:]
    h = (g * jax.nn.sigmoid(g) * u).astype(_BF16)
    wdn = wdn_ref[0].astype(_BF16)
    eo = lax.dot_general(h, wdn, (((1,), (1,)), ((), ())),
                         preferred_element_type=_F32)

    rows = lax.broadcasted_iota(_I32, (BLK, 1), 0) + blk * BLK
    maskv = jnp.logical_and(rows >= lo, rows < hi)
    contrib = jnp.where(maskv, eo, 0.0)

    @pl.when(first)
    def _():
        rs_ref[...] = contrib

    @pl.when(jnp.logical_not(first))
    def _():
        rs_ref[...] = rs_ref[...] + contrib


def _run_grouped(meta, xs, w_gate_up, w_down):
    grid_spec = pltpu.PrefetchScalarGridSpec(
        num_scalar_prefetch=1,
        grid=(NSTEPS,),
        in_specs=[
            pl.BlockSpec((BLK, D), lambda s, m: (m[1, s], 0)),
            pl.BlockSpec((1, 2 * FF, D), lambda s, m: (m[0, s], 0, 0)),
            pl.BlockSpec((1, D, FF), lambda s, m: (m[0, s], 0, 0)),
        ],
        out_specs=pl.BlockSpec((BLK, D), lambda s, m: (m[1, s], 0)),
    )
    return pl.pallas_call(
        _grouped_body,
        grid_spec=grid_spec,
        out_shape=jax.ShapeDtypeStruct((TK, D), _F32),
    )(meta, xs, w_gate_up, w_down)


def _make_sc_gather():
    mesh = plsc.VectorSubcoreMesh(core_axis_name="c", subcore_axis_name="s")
    NW = 32
    TPW = T // NW

    @functools.partial(
        pl.kernel, mesh=mesh,
        out_type=(
            jax.ShapeDtypeStruct((T, D), _F32),
            jax.ShapeDtypeStruct((T, D), _F32),
        ),
        scratch_types=[
            pltpu.VMEM((TPW,), _I32),
            pltpu.VMEM((TPW, D), _F32),
            pltpu.SemaphoreType.DMA,
        ],
    )
    def sc_gather(rs_hbm, p0_hbm, p1_hbm, a_hbm, b_hbm, idx_v, rows_v, sem):
        wid = lax.axis_index("s") * 2 + lax.axis_index("c")
        base = wid * TPW
        pltpu.sync_copy(p0_hbm.at[pl.ds(base, TPW)], idx_v)
        pltpu.async_copy(rs_hbm.at[idx_v], rows_v, sem).wait()
        pltpu.sync_copy(rows_v, a_hbm.at[pl.ds(base, TPW)])
        pltpu.sync_copy(p1_hbm.at[pl.ds(base, TPW)], idx_v)
        pltpu.async_copy(rs_hbm.at[idx_v], rows_v, sem).wait()
        pltpu.sync_copy(rows_v, b_hbm.at[pl.ds(base, TPW)])

    return sc_gather


def _get_sc_gather():
    if "gather" not in _SC_CACHE:
        _SC_CACHE["gather"] = _make_sc_gather()
    return _SC_CACHE["gather"]


def _shared_body(x_ref, sgu_ref, sdn_ref, a_ref, b_ref, w1_ref, w2_ref,
                 out_ref, sgu_c, sdn_c):
    @pl.when(pl.program_id(0) == 0)
    def _():
        sgu_c[...] = sgu_ref[...].astype(_BF16)
        sdn_c[...] = sdn_ref[...].astype(_BF16)

    xb = x_ref[...].astype(_BF16)
    sgu = lax.dot_general(xb, sgu_c[...], (((1,), (1,)), ((), ())),
                          preferred_element_type=_F32)
    sg = sgu[:, :SF]
    su = sgu[:, SF:]
    sh = (sg * jax.nn.sigmoid(sg) * su).astype(_BF16)
    out = lax.dot_general(sh, sdn_c[...], (((1,), (1,)), ((), ())),
                          preferred_element_type=_F32)
    out_ref[...] = out + w1_ref[...] * a_ref[...] + w2_ref[...] * b_ref[...]


def _run_shared(x, shared_gate_up, shared_down, a, b, w1, w2):
    return pl.pallas_call(
        _shared_body,
        grid=(T // BT2,),
        in_specs=[
            pl.BlockSpec((BT2, D), lambda i: (i, 0)),
            pl.BlockSpec((2 * SF, D), lambda i: (0, 0)),
            pl.BlockSpec((D, SF), lambda i: (0, 0)),
            pl.BlockSpec((BT2, D), lambda i: (i, 0)),
            pl.BlockSpec((BT2, D), lambda i: (i, 0)),
            pl.BlockSpec((BT2, 1), lambda i: (i, 0)),
            pl.BlockSpec((BT2, 1), lambda i: (i, 0)),
        ],
        out_specs=pl.BlockSpec((BT2, D), lambda i: (i, 0)),
        out_shape=jax.ShapeDtypeStruct((T, D), _F32),
        scratch_shapes=[
            pltpu.VMEM((2 * SF, D), _BF16),
            pltpu.VMEM((D, SF), _BF16),
        ],
    )(x, shared_gate_up, shared_down, a, b, w1, w2)


def kernel(hidden_states, gate_w, expert_bias, w_gate_up, w_down,
           shared_gate_up, shared_down):
    orig_shape = hidden_states.shape
    x = hidden_states.reshape(-1, orig_shape[-1])
    bias2 = expert_bias.reshape(1, E)

    pos2, p0, p1, w1, w2, meta = _run_router(x, gate_w, bias2)
    pos_flat = pos2.reshape(TK)

    xs = _get_sc_dispatch()(x, pos_flat)

    rs = _run_grouped(meta, xs, w_gate_up, w_down)

    a, b = _get_sc_gather()(rs, p0.reshape(T), p1.reshape(T))

    out = _run_shared(x, shared_gate_up, shared_down, a, b, w1, w2)
    return out.reshape(orig_shape)


# speedup vs baseline: 1.1000x; 1.0392x over previous
import functools

import jax
import jax.numpy as jnp
from jax import lax
from jax.experimental import pallas as pl
from jax.experimental.pallas import tpu as pltpu
from jax.experimental.pallas import tpu_sc as plsc

T = 2048
D = 1024
E = 8
FF = 1024
SF = 1024
TK = 2 * T
BLK = 256
NBP = TK // BLK + E - 1
NSTEPS = NBP
TKP = NBP * BLK
BT2 = 256

_F32 = jnp.float32
_BF16 = jnp.bfloat16
_I32 = jnp.int32


def _router_body(x_ref, gw_ref, bias_ref, pos_ref, p0_ref, p1_ref,
                 w1_ref, w2_ref, meta_ref):
    x = x_ref[...]
    logits = lax.dot_general(
        x, gw_ref[...], (((1,), (1,)), ((), ())),
        preferred_element_type=_F32,
        precision=lax.Precision.DEFAULT,
    )
    scores = jax.nn.sigmoid(logits)
    sfc = scores + bias_ref[...]

    lane = lax.broadcasted_iota(_I32, (T, E), 1)
    big = _F32(1e30)

    m1 = jnp.max(sfc, axis=1, keepdims=True)
    i1 = jnp.min(jnp.where(sfc >= m1, lane, E), axis=1, keepdims=True)
    oh1 = lane == i1
    sfc2 = jnp.where(oh1, -big, sfc)
    m2 = jnp.max(sfc2, axis=1, keepdims=True)
    i2 = jnp.min(jnp.where(sfc2 >= m2, lane, E), axis=1, keepdims=True)
    oh2 = lane == i2

    w1 = jnp.sum(jnp.where(oh1, scores, 0.0), axis=1, keepdims=True)
    w2 = jnp.sum(jnp.where(oh2, scores, 0.0), axis=1, keepdims=True)
    norm = w1 + w2 + 1e-20
    w1 = w1 / norm
    w2 = w2 / norm

    oh_f = oh1.astype(_F32) + oh2.astype(_F32)
    oh_b = oh_f.astype(_BF16)

    CH = 128
    tri = (lax.broadcasted_iota(_I32, (CH, CH), 0)
           > lax.broadcasted_iota(_I32, (CH, CH), 1)).astype(_BF16)
    chunks = []
    running = jnp.zeros((1, E), _F32)
    for c in range(T // CH):
        blk = oh_b[c * CH:(c + 1) * CH, :]
        within = lax.dot_general(
            tri, blk, (((1,), (0,)), ((), ())), preferred_element_type=_F32)
        chunks.append(within + running)
        running = running + jnp.sum(blk.astype(_F32), axis=0, keepdims=True)
    cexcl = jnp.concatenate(chunks, axis=0)
    counts = running

    cb = jnp.floor((counts + (BLK - 1.0)) * (1.0 / BLK))
    cb_b = jnp.broadcast_to(cb * BLK, (T, E))
    off1 = jnp.sum(jnp.where(lane < i1, cb_b, 0.0), axis=1, keepdims=True)
    off2 = jnp.sum(jnp.where(lane < i2, cb_b, 0.0), axis=1, keepdims=True)
    rank1 = jnp.sum(jnp.where(oh1, cexcl, 0.0), axis=1, keepdims=True)
    rank2 = jnp.sum(jnp.where(oh2, cexcl, 0.0), axis=1, keepdims=True)
    pos1 = (off1 + rank1).astype(_I32)
    pos2 = (off2 + rank2).astype(_I32)

    pos_ref[...] = jnp.concatenate([pos1, pos2], axis=1)
    p0_ref[...] = pos1
    p1_ref[...] = pos2
    w1_ref[...] = w1
    w2_ref[...] = w2

    ones_col = jnp.ones((T, 1), _BF16)
    counts_col = lax.dot_general(
        oh_b, ones_col, (((0,), (0,)), ((), ())),
        preferred_element_type=_F32)
    ltri = (lax.broadcasted_iota(_I32, (E, E), 1)
            < lax.broadcasted_iota(_I32, (E, E), 0)).astype(_BF16)
    cb_col = jnp.floor((counts_col + (BLK - 1.0)) * (1.0 / BLK))
    start = lax.dot_general(
        ltri, cb_col.astype(_BF16), (((1,), (0,)), ((), ())),
        preferred_element_type=_F32).astype(_I32)
    total = jnp.sum(cb_col).astype(_I32)

    svec = lax.broadcasted_iota(_I32, (E, 128), 1)
    s_eff = jnp.minimum(svec, total - 1)
    start_b2 = jnp.broadcast_to(start, (E, 128))
    e_of_s = jnp.sum((start_b2 <= s_eff).astype(_I32), axis=0,
                     keepdims=True) - 1
    blk_s = s_eff[0:1, :]
    valid = (svec[0:1, :] < total).astype(_I32)

    meta_ref[...] = jnp.concatenate(
        [e_of_s, blk_s, valid, jnp.zeros((5, 128), _I32)], axis=0)


def _run_router(x, gate_w, bias2):
    return pl.pallas_call(
        _router_body,
        out_shape=(
            jax.ShapeDtypeStruct((T, 2), _I32),
            jax.ShapeDtypeStruct((T, 1), _I32),
            jax.ShapeDtypeStruct((T, 1), _I32),
            jax.ShapeDtypeStruct((T, 1), _F32),
            jax.ShapeDtypeStruct((T, 1), _F32),
            jax.ShapeDtypeStruct((8, 128), _I32),
        ),
    )(x, gate_w, bias2)


def _make_sc_dispatch():
    mesh = plsc.VectorSubcoreMesh(core_axis_name="c", subcore_axis_name="s")
    NW = 32
    CHUNK = TK // NW
    SUB = CHUNK // 2

    @functools.partial(
        pl.kernel, mesh=mesh,
        out_type=jax.ShapeDtypeStruct((TKP, D), _F32),
        scratch_types=[
            pltpu.VMEM((SUB,), _I32),
            pltpu.VMEM((SUB,), _I32),
            pltpu.VMEM((SUB, D), _F32),
            pltpu.SemaphoreType.DMA,
        ],
    )
    def sc_dispatch(x_hbm, pos_hbm, xs_hbm, pos_v, tok_v, rows_v, sem):
        wid = lax.axis_index("s") * 2 + lax.axis_index("c")
        base = wid * CHUNK
        for sub in range(2):
            sbase = base + sub * SUB
            pltpu.sync_copy(pos_hbm.at[pl.ds(sbase, SUB)], pos_v)
            for j in range(SUB // 16):
                t16 = lax.shift_right_logical(
                    lax.iota(_I32, 16), 1) + ((sbase + 16 * j) // 2)
                tok_v[pl.ds(16 * j, 16)] = t16
            pltpu.async_copy(x_hbm.at[tok_v], rows_v, sem).wait()
            pltpu.async_copy(rows_v, xs_hbm.at[pos_v], sem).wait()

    return sc_dispatch


_SC_CACHE = {}


def _get_sc_dispatch():
    if "dispatch" not in _SC_CACHE:
        _SC_CACHE["dispatch"] = _make_sc_dispatch()
    return _SC_CACHE["dispatch"]


def _grouped_body(meta_ref, xs_ref, wgu_ref, wdn_ref, rs_ref):
    s = pl.program_id(0)
    valid = meta_ref[2, s] == 1

    @pl.when(valid)
    def _():
        xb = xs_ref[...].astype(_BF16)
        wgu = wgu_ref[0].astype(_BF16)
        gu = lax.dot_general(xb, wgu, (((1,), (1,)), ((), ())),
                             preferred_element_type=_F32)
        g = gu[:, :FF]
        u = gu[:, FF:]
        h = (g * jax.nn.sigmoid(g) * u).astype(_BF16)
        wdn = wdn_ref[0].astype(_BF16)
        eo = lax.dot_general(h, wdn, (((1,), (1,)), ((), ())),
                             preferred_element_type=_F32)
        rs_ref[...] = eo


def _run_grouped(meta, xs, w_gate_up, w_down):
    grid_spec = pltpu.PrefetchScalarGridSpec(
        num_scalar_prefetch=1,
        grid=(NSTEPS,),
        in_specs=[
            pl.BlockSpec((BLK, D), lambda s, m: (m[1, s], 0)),
            pl.BlockSpec((1, 2 * FF, D), lambda s, m: (m[0, s], 0, 0)),
            pl.BlockSpec((1, D, FF), lambda s, m: (m[0, s], 0, 0)),
        ],
        out_specs=pl.BlockSpec((BLK, D), lambda s, m: (m[1, s], 0)),
    )
    return pl.pallas_call(
        _grouped_body,
        grid_spec=grid_spec,
        out_shape=jax.ShapeDtypeStruct((TKP, D), _F32),
    )(meta, xs, w_gate_up, w_down)


def _make_sc_gather():
    mesh = plsc.VectorSubcoreMesh(core_axis_name="c", subcore_axis_name="s")
    NW = 32
    TPW = T // NW

    @functools.partial(
        pl.kernel, mesh=mesh,
        out_type=(
            jax.ShapeDtypeStruct((T, D), _F32),
            jax.ShapeDtypeStruct((T, D), _F32),
        ),
        scratch_types=[
            pltpu.VMEM((TPW,), _I32),
            pltpu.VMEM((TPW, D), _F32),
            pltpu.SemaphoreType.DMA,
        ],
    )
    def sc_gather(rs_hbm, p0_hbm, p1_hbm, a_hbm, b_hbm, idx_v, rows_v, sem):
        wid = lax.axis_index("s") * 2 + lax.axis_index("c")
        base = wid * TPW
        pltpu.sync_copy(p0_hbm.at[pl.ds(base, TPW)], idx_v)
        pltpu.async_copy(rs_hbm.at[idx_v], rows_v, sem).wait()
        pltpu.sync_copy(rows_v, a_hbm.at[pl.ds(base, TPW)])
        pltpu.sync_copy(p1_hbm.at[pl.ds(base, TPW)], idx_v)
        pltpu.async_copy(rs_hbm.at[idx_v], rows_v, sem).wait()
        pltpu.sync_copy(rows_v, b_hbm.at[pl.ds(base, TPW)])

    return sc_gather


def _get_sc_gather():
    if "gather" not in _SC_CACHE:
        _SC_CACHE["gather"] = _make_sc_gather()
    return _SC_CACHE["gather"]


def _shared_body(x_ref, sgu_ref, sdn_ref, a_ref, b_ref, w1_ref, w2_ref,
                 out_ref, sgu_c, sdn_c):
    @pl.when(pl.program_id(0) == 0)
    def _():
        sgu_c[...] = sgu_ref[...].astype(_BF16)
        sdn_c[...] = sdn_ref[...].astype(_BF16)

    xb = x_ref[...].astype(_BF16)
    sgu = lax.dot_general(xb, sgu_c[...], (((1,), (1,)), ((), ())),
                          preferred_element_type=_F32)
    sg = sgu[:, :SF]
    su = sgu[:, SF:]
    sh = (sg * jax.nn.sigmoid(sg) * su).astype(_BF16)
    out = lax.dot_general(sh, sdn_c[...], (((1,), (1,)), ((), ())),
                          preferred_element_type=_F32)
    out_ref[...] = out + w1_ref[...] * a_ref[...] + w2_ref[...] * b_ref[...]


def _run_shared(x, shared_gate_up, shared_down, a, b, w1, w2):
    return pl.pallas_call(
        _shared_body,
        grid=(T // BT2,),
        in_specs=[
            pl.BlockSpec((BT2, D), lambda i: (i, 0)),
            pl.BlockSpec((2 * SF, D), lambda i: (0, 0)),
            pl.BlockSpec((D, SF), lambda i: (0, 0)),
            pl.BlockSpec((BT2, D), lambda i: (i, 0)),
            pl.BlockSpec((BT2, D), lambda i: (i, 0)),
            pl.BlockSpec((BT2, 1), lambda i: (i, 0)),
            pl.BlockSpec((BT2, 1), lambda i: (i, 0)),
        ],
        out_specs=pl.BlockSpec((BT2, D), lambda i: (i, 0)),
        out_shape=jax.ShapeDtypeStruct((T, D), _F32),
        scratch_shapes=[
            pltpu.VMEM((2 * SF, D), _BF16),
            pltpu.VMEM((D, SF), _BF16),
        ],
    )(x, shared_gate_up, shared_down, a, b, w1, w2)


def kernel(hidden_states, gate_w, expert_bias, w_gate_up, w_down,
           shared_gate_up, shared_down):
    orig_shape = hidden_states.shape
    x = hidden_states.reshape(-1, orig_shape[-1])
    bias2 = expert_bias.reshape(1, E)

    pos2, p0, p1, w1, w2, meta = _run_router(x, gate_w, bias2)
    pos_flat = pos2.reshape(TK)

    xs = _get_sc_dispatch()(x, pos_flat)

    rs = _run_grouped(meta, xs, w_gate_up, w_down)

    a, b = _get_sc_gather()(rs, p0.reshape(T), p1.reshape(T))

    out = _run_shared(x, shared_gate_up, shared_down, a, b, w1, w2)
    return out.reshape(orig_shape)


# speedup vs baseline: 1.1044x; 1.0040x over previous
import functools

import jax
import jax.numpy as jnp
from jax import lax
from jax.experimental import pallas as pl
from jax.experimental.pallas import tpu as pltpu
from jax.experimental.pallas import tpu_sc as plsc

T = 2048
D = 1024
E = 8
FF = 1024
SF = 1024
TK = 2 * T
BLK = 256
NBP = TK // BLK + E - 1
NSTEPS = NBP
TKP = NBP * BLK
BT2 = 256

_F32 = jnp.float32
_BF16 = jnp.bfloat16
_I32 = jnp.int32


def _router_body(x_ref, gw_ref, bias_ref, pos_ref, p0_ref, p1_ref,
                 w1_ref, w2_ref, meta_ref):
    x = x_ref[...]
    logits = lax.dot_general(
        x, gw_ref[...], (((1,), (1,)), ((), ())),
        preferred_element_type=_F32,
        precision=lax.Precision.DEFAULT,
    )
    scores = jax.nn.sigmoid(logits)
    sfc = scores + bias_ref[...]

    lane = lax.broadcasted_iota(_I32, (T, E), 1)
    big = _F32(1e30)

    m1 = jnp.max(sfc, axis=1, keepdims=True)
    i1 = jnp.min(jnp.where(sfc >= m1, lane, E), axis=1, keepdims=True)
    oh1 = lane == i1
    sfc2 = jnp.where(oh1, -big, sfc)
    m2 = jnp.max(sfc2, axis=1, keepdims=True)
    i2 = jnp.min(jnp.where(sfc2 >= m2, lane, E), axis=1, keepdims=True)
    oh2 = lane == i2

    w1 = jnp.sum(jnp.where(oh1, scores, 0.0), axis=1, keepdims=True)
    w2 = jnp.sum(jnp.where(oh2, scores, 0.0), axis=1, keepdims=True)
    norm = w1 + w2 + 1e-20
    w1 = w1 / norm
    w2 = w2 / norm

    oh_f = oh1.astype(_F32) + oh2.astype(_F32)
    oh_b = oh_f.astype(_BF16)

    CH = 128
    tri = (lax.broadcasted_iota(_I32, (CH, CH), 0)
           > lax.broadcasted_iota(_I32, (CH, CH), 1)).astype(_BF16)
    chunks = []
    running = jnp.zeros((1, E), _F32)
    for c in range(T // CH):
        blk = oh_b[c * CH:(c + 1) * CH, :]
        within = lax.dot_general(
            tri, blk, (((1,), (0,)), ((), ())), preferred_element_type=_F32)
        chunks.append(within + running)
        running = running + jnp.sum(blk.astype(_F32), axis=0, keepdims=True)
    cexcl = jnp.concatenate(chunks, axis=0)
    counts = running

    cb = jnp.floor((counts + (BLK - 1.0)) * (1.0 / BLK))
    cb_b = jnp.broadcast_to(cb * BLK, (T, E))
    off1 = jnp.sum(jnp.where(lane < i1, cb_b, 0.0), axis=1, keepdims=True)
    off2 = jnp.sum(jnp.where(lane < i2, cb_b, 0.0), axis=1, keepdims=True)
    rank1 = jnp.sum(jnp.where(oh1, cexcl, 0.0), axis=1, keepdims=True)
    rank2 = jnp.sum(jnp.where(oh2, cexcl, 0.0), axis=1, keepdims=True)
    pos1 = (off1 + rank1).astype(_I32)
    pos2 = (off2 + rank2).astype(_I32)

    pos_ref[...] = jnp.concatenate([pos1, pos2], axis=1)
    p0_ref[...] = pos1
    p1_ref[...] = pos2
    w1_ref[...] = w1
    w2_ref[...] = w2

    ones_col = jnp.ones((T, 1), _BF16)
    counts_col = lax.dot_general(
        oh_b, ones_col, (((0,), (0,)), ((), ())),
        preferred_element_type=_F32)
    ltri = (lax.broadcasted_iota(_I32, (E, E), 1)
            < lax.broadcasted_iota(_I32, (E, E), 0)).astype(_BF16)
    cb_col = jnp.floor((counts_col + (BLK - 1.0)) * (1.0 / BLK))
    start = lax.dot_general(
        ltri, cb_col.astype(_BF16), (((1,), (0,)), ((), ())),
        preferred_element_type=_F32).astype(_I32)
    total = jnp.sum(cb_col).astype(_I32)

    svec = lax.broadcasted_iota(_I32, (E, 128), 1)
    s_eff = jnp.minimum(svec, total - 1)
    start_b2 = jnp.broadcast_to(start, (E, 128))
    e_of_s = jnp.sum((start_b2 <= s_eff).astype(_I32), axis=0,
                     keepdims=True) - 1
    blk_s = s_eff[0:1, :]
    valid = (svec[0:1, :] < total).astype(_I32)

    meta_ref[...] = jnp.concatenate(
        [e_of_s, blk_s, valid, jnp.zeros((5, 128), _I32)], axis=0)


def _run_router(x, gate_w, bias2):
    return pl.pallas_call(
        _router_body,
        out_shape=(
            jax.ShapeDtypeStruct((T, 2), _I32),
            jax.ShapeDtypeStruct((T, 1), _I32),
            jax.ShapeDtypeStruct((T, 1), _I32),
            jax.ShapeDtypeStruct((T, 1), _F32),
            jax.ShapeDtypeStruct((T, 1), _F32),
            jax.ShapeDtypeStruct((8, 128), _I32),
        ),
    )(x, gate_w, bias2)


def _make_sc_dispatch():
    mesh = plsc.VectorSubcoreMesh(core_axis_name="c", subcore_axis_name="s")
    NW = 32
    CHUNK = TK // NW
    SUB = CHUNK // 2

    @functools.partial(
        pl.kernel, mesh=mesh,
        out_type=jax.ShapeDtypeStruct((TKP, D), _F32),
        scratch_types=[
            pltpu.VMEM((SUB,), _I32),
            pltpu.VMEM((SUB,), _I32),
            pltpu.VMEM((SUB, D), _F32),
            pltpu.SemaphoreType.DMA,
        ],
    )
    def sc_dispatch(x_hbm, pos_hbm, xs_hbm, pos_v, tok_v, rows_v, sem):
        wid = lax.axis_index("s") * 2 + lax.axis_index("c")
        base = wid * CHUNK
        for sub in range(2):
            sbase = base + sub * SUB
            pltpu.sync_copy(pos_hbm.at[pl.ds(sbase, SUB)], pos_v)
            for j in range(SUB // 16):
                t16 = lax.shift_right_logical(
                    lax.iota(_I32, 16), 1) + ((sbase + 16 * j) // 2)
                tok_v[pl.ds(16 * j, 16)] = t16
            pltpu.async_copy(x_hbm.at[tok_v], rows_v, sem).wait()
            pltpu.async_copy(rows_v, xs_hbm.at[pos_v], sem).wait()

    return sc_dispatch


_SC_CACHE = {}


def _get_sc_dispatch():
    if "dispatch" not in _SC_CACHE:
        _SC_CACHE["dispatch"] = _make_sc_dispatch()
    return _SC_CACHE["dispatch"]


def _grouped_body(meta_ref, xs_ref, wgu_ref, wdn_ref, rs_ref):
    s = pl.program_id(0)
    valid = meta_ref[2, s] == 1

    @pl.when(valid)
    def _():
        xb = xs_ref[...].astype(_BF16)
        wgu = wgu_ref[0].astype(_BF16)
        gu = lax.dot_general(xb, wgu, (((1,), (1,)), ((), ())),
                             preferred_element_type=_F32)
        g = gu[:, :FF]
        u = gu[:, FF:]
        h = (g * jax.nn.sigmoid(g) * u).astype(_BF16)
        wdn = wdn_ref[0].astype(_BF16)
        eo = lax.dot_general(h, wdn, (((1,), (1,)), ((), ())),
                             preferred_element_type=_F32)
        rs_ref[...] = eo


def _run_grouped(meta, xs, w_gate_up, w_down):
    grid_spec = pltpu.PrefetchScalarGridSpec(
        num_scalar_prefetch=1,
        grid=(NSTEPS,),
        in_specs=[
            pl.BlockSpec((BLK, D), lambda s, m: (m[1, s], 0)),
            pl.BlockSpec((1, 2 * FF, D), lambda s, m: (m[0, s], 0, 0)),
            pl.BlockSpec((1, D, FF), lambda s, m: (m[0, s], 0, 0)),
        ],
        out_specs=pl.BlockSpec((BLK, D), lambda s, m: (m[1, s], 0)),
    )
    return pl.pallas_call(
        _grouped_body,
        grid_spec=grid_spec,
        out_shape=jax.ShapeDtypeStruct((TKP, D), _F32),
    )(meta, xs, w_gate_up, w_down)


def _make_sc_gather():
    mesh = plsc.VectorSubcoreMesh(core_axis_name="c", subcore_axis_name="s")
    NW = 32
    TPW = T // NW

    @functools.partial(
        pl.kernel, mesh=mesh,
        out_type=(
            jax.ShapeDtypeStruct((T, D), _F32),
            jax.ShapeDtypeStruct((T, D), _F32),
        ),
        scratch_types=[
            pltpu.VMEM((TPW,), _I32),
            pltpu.VMEM((TPW, D), _F32),
            pltpu.SemaphoreType.DMA,
        ],
    )
    def sc_gather(rs_hbm, p0_hbm, p1_hbm, a_hbm, b_hbm, idx_v, rows_v, sem):
        wid = lax.axis_index("s") * 2 + lax.axis_index("c")
        base = wid * TPW
        pltpu.sync_copy(p0_hbm.at[pl.ds(base, TPW)], idx_v)
        pltpu.async_copy(rs_hbm.at[idx_v], rows_v, sem).wait()
        pltpu.sync_copy(rows_v, a_hbm.at[pl.ds(base, TPW)])
        pltpu.sync_copy(p1_hbm.at[pl.ds(base, TPW)], idx_v)
        pltpu.async_copy(rs_hbm.at[idx_v], rows_v, sem).wait()
        pltpu.sync_copy(rows_v, b_hbm.at[pl.ds(base, TPW)])

    return sc_gather


def _get_sc_gather():
    if "gather" not in _SC_CACHE:
        _SC_CACHE["gather"] = _make_sc_gather()
    return _SC_CACHE["gather"]


def _shared_body(x_ref, sgu_ref, sdn_ref, out_ref, sgu_c, sdn_c):
    @pl.when(pl.program_id(0) == 0)
    def _():
        sgu_c[...] = sgu_ref[...].astype(_BF16)
        sdn_c[...] = sdn_ref[...].astype(_BF16)

    xb = x_ref[...].astype(_BF16)
    sgu = lax.dot_general(xb, sgu_c[...], (((1,), (1,)), ((), ())),
                          preferred_element_type=_F32)
    sg = sgu[:, :SF]
    su = sgu[:, SF:]
    sh = (sg * jax.nn.sigmoid(sg) * su).astype(_BF16)
    out_ref[...] = lax.dot_general(sh, sdn_c[...], (((1,), (1,)), ((), ())),
                                   preferred_element_type=_F32)


def _run_shared(x, shared_gate_up, shared_down):
    return pl.pallas_call(
        _shared_body,
        grid=(T // BT2,),
        in_specs=[
            pl.BlockSpec((BT2, D), lambda i: (i, 0)),
            pl.BlockSpec((2 * SF, D), lambda i: (0, 0)),
            pl.BlockSpec((D, SF), lambda i: (0, 0)),
        ],
        out_specs=pl.BlockSpec((BT2, D), lambda i: (i, 0)),
        out_shape=jax.ShapeDtypeStruct((T, D), _F32),
        scratch_shapes=[
            pltpu.VMEM((2 * SF, D), _BF16),
            pltpu.VMEM((D, SF), _BF16),
        ],
    )(x, shared_gate_up, shared_down)


def _combine_body(sh_ref, a_ref, b_ref, w1_ref, w2_ref, out_ref):
    out_ref[...] = (sh_ref[...] + w1_ref[...] * a_ref[...]
                    + w2_ref[...] * b_ref[...])


def _run_combine(sh, a, b, w1, w2):
    return pl.pallas_call(
        _combine_body,
        grid=(T // BT2,),
        in_specs=[
            pl.BlockSpec((BT2, D), lambda i: (i, 0)),
            pl.BlockSpec((BT2, D), lambda i: (i, 0)),
            pl.BlockSpec((BT2, D), lambda i: (i, 0)),
            pl.BlockSpec((BT2, 1), lambda i: (i, 0)),
            pl.BlockSpec((BT2, 1), lambda i: (i, 0)),
        ],
        out_specs=pl.BlockSpec((BT2, D), lambda i: (i, 0)),
        out_shape=jax.ShapeDtypeStruct((T, D), _F32),
    )(sh, a, b, w1, w2)


def kernel(hidden_states, gate_w, expert_bias, w_gate_up, w_down,
           shared_gate_up, shared_down):
    orig_shape = hidden_states.shape
    x = hidden_states.reshape(-1, orig_shape[-1])
    bias2 = expert_bias.reshape(1, E)

    pos2, p0, p1, w1, w2, meta = _run_router(x, gate_w, bias2)
    pos_flat = pos2.reshape(TK)

    xs = _get_sc_dispatch()(x, pos_flat)

    rs = _run_grouped(meta, xs, w_gate_up, w_down)

    sh = _run_shared(x, shared_gate_up, shared_down)
    a, b = _get_sc_gather()(rs, p0.reshape(T), p1.reshape(T))

    out = _run_combine(sh, a, b, w1, w2)
    return out.reshape(orig_shape)
